# baseline (speedup 1.0000x reference)
import functools

import jax
import jax.numpy as jnp
from jax import lax
from jax.experimental import pallas as pl
from jax.experimental.pallas import tpu as pltpu
from jax.experimental.pallas import tpu_sc as plsc

N = 10000
HID = 128
E = 320000
EL = 100000
NW = 32
TILES = 16
CHUNK = 128
KCH = 80
EDGES_T = KCH * CHUNK
EP = NW * EDGES_T
NACC = 10240
ROWS_T = NACC // TILES
DEGW = 16
DCHUNK = 128
KD = 25
ELP_T = KD * DCHUNK
ELP = NW * ELP_T

_sc_mesh = plsc.VectorSubcoreMesh(core_axis_name="c", subcore_axis_name="s")


HALF = KCH // 2
PIPE = HALF // 2


@functools.partial(
    pl.kernel, mesh=_sc_mesh,
    out_type=jax.ShapeDtypeStruct((2, NACC, HID), jnp.float32),
    scratch_types=[
        pltpu.VMEM((HALF, CHUNK), jnp.int32),
        pltpu.VMEM((HALF, CHUNK), jnp.int32),
        pltpu.VMEM((CHUNK, HID), jnp.float32),
        pltpu.VMEM((CHUNK, HID), jnp.float32),
        pltpu.VMEM_SHARED((NACC, HID), jnp.float32),
        pltpu.SemaphoreType.DMA,
        pltpu.SemaphoreType.DMA,
    ],
)
def _spmm(p_h, src_h, dst_h, zrow_h, part_h, srcv, dstv, rows0, rows1,
          accsp, sem0, sem1):
    c = lax.axis_index("c")
    s = lax.axis_index("s")
    eb = c * TILES + s
    pltpu.sync_copy(zrow_h, accsp.at[pl.ds(s * ROWS_T, ROWS_T)])
    plsc.subcore_barrier()

    for half in (0, 1):
        pltpu.sync_copy(src_h.at[eb, pl.ds(half * HALF, HALF)], srcv)
        pltpu.sync_copy(dst_h.at[eb, pl.ds(half * HALF, HALF)], dstv)
        pltpu.async_copy(p_h.at[srcv.at[0]], rows0, sem0)

        def body(j2, carry):
            pltpu.async_copy(p_h.at[srcv.at[2 * j2 + 1]], rows1, sem1)
            pltpu.make_async_copy(p_h.at[srcv.at[0]], rows0, sem0).wait()
            pltpu.sync_copy(rows0, accsp.at[dstv.at[2 * j2]], add=True)

            @pl.when(j2 < PIPE - 1)
            def _prefetch():
                pltpu.async_copy(p_h.at[srcv.at[2 * j2 + 2]], rows0, sem0)

            pltpu.make_async_copy(p_h.at[srcv.at[0]], rows1, sem1).wait()
            pltpu.sync_copy(rows1, accsp.at[dstv.at[2 * j2 + 1]], add=True)
            return carry

        lax.fori_loop(0, PIPE, body, 0)

    plsc.subcore_barrier()
    pltpu.sync_copy(accsp.at[pl.ds(s * ROWS_T, ROWS_T)],
                    part_h.at[c, pl.ds(s * ROWS_T, ROWS_T)])


@functools.partial(
    pl.kernel, mesh=_sc_mesh,
    out_type=jax.ShapeDtypeStruct((2, NACC, DEGW), jnp.float32),
    compiler_params=pltpu.CompilerParams(use_tc_tiling_on_sc=False),
    scratch_types=[
        pltpu.VMEM((KCH, CHUNK), jnp.int32),
        pltpu.VMEM((CHUNK, DEGW), jnp.float32),
        pltpu.VMEM_SHARED((NACC, DEGW), jnp.float32),
    ],
)
def _degree(dst_h, zdeg_h, ones_h, dego_h, dstv, onesv, degsp):
    c = lax.axis_index("c")
    s = lax.axis_index("s")
    eb = c * TILES + s
    pltpu.sync_copy(dst_h.at[eb], dstv)
    pltpu.sync_copy(ones_h, onesv)
    pltpu.sync_copy(zdeg_h, degsp.at[pl.ds(s * ROWS_T, ROWS_T)])
    plsc.subcore_barrier()

    def body(j, carry):
        pltpu.sync_copy(onesv, degsp.at[dstv.at[j]], add=True)
        return carry

    lax.fori_loop(0, KCH, body, 0)
    plsc.subcore_barrier()
    pltpu.sync_copy(degsp.at[pl.ds(s * ROWS_T, ROWS_T)],
                    dego_h.at[c, pl.ds(s * ROWS_T, ROWS_T)])


KDP = KD // 2


OROWS = ELP_T * DEGW // HID


@functools.partial(
    pl.kernel, mesh=_sc_mesh,
    out_type=jax.ShapeDtypeStruct((NW, ELP_T, DEGW), jnp.float32),
    compiler_params=pltpu.CompilerParams(use_tc_tiling_on_sc=False),
    scratch_types=[
        pltpu.VMEM((KD, DCHUNK), jnp.int32),
        pltpu.VMEM((KD, DCHUNK), jnp.int32),
        pltpu.VMEM((ELP_T, DEGW), jnp.float32),
        pltpu.SemaphoreType.DMA,
        pltpu.SemaphoreType.DMA,
    ],
)
def _decode(u_h, v_h, el0_h, el1_h, out_h, i0v, i1v, outv, sem0, sem1):
    c = lax.axis_index("c")
    s = lax.axis_index("s")
    wid = c * TILES + s
    pltpu.sync_copy(el0_h.at[wid], i0v)
    pltpu.sync_copy(el1_h.at[wid], i1v)

    def oslice(j):
        return outv.at[pl.ds(j * DCHUNK, DCHUNK)]

    pltpu.async_copy(u_h.at[i0v.at[0]], oslice(0), sem0)

    def body(j2, carry):
        pltpu.async_copy(u_h.at[i0v.at[2 * j2 + 1]], oslice(2 * j2 + 1), sem1)
        pltpu.make_async_copy(u_h.at[i0v.at[0]], oslice(2 * j2), sem0).wait()
        pltpu.sync_copy(v_h.at[i1v.at[2 * j2]], oslice(2 * j2), add=True)
        pltpu.async_copy(u_h.at[i0v.at[2 * j2 + 2]], oslice(2 * j2 + 2), sem0)
        pltpu.make_async_copy(u_h.at[i0v.at[0]], oslice(2 * j2 + 1), sem1).wait()
        pltpu.sync_copy(v_h.at[i1v.at[2 * j2 + 1]], oslice(2 * j2 + 1), add=True)
        return carry

    lax.fori_loop(0, KDP, body, 0)
    pltpu.make_async_copy(u_h.at[i0v.at[0]], oslice(KD - 1), sem0).wait()
    pltpu.sync_copy(v_h.at[i1v.at[KD - 1]], oslice(KD - 1), add=True)
    pltpu.sync_copy(outv, out_h.at[wid])


CBLK = 1024


def _compact_body(pin_ref, out_ref):
    x = pin_ref[...]
    z = x.reshape(CBLK // 8, 8, DEGW)[:, :, 0]
    out_ref[...] = z.reshape(CBLK)


def _compact(pred16):
    return pl.pallas_call(
        _compact_body,
        grid=(ELP // CBLK,),
        in_specs=[pl.BlockSpec((CBLK // 8, HID), lambda t: (t, 0))],
        out_specs=pl.BlockSpec((CBLK,), lambda t: (t,)),
        out_shape=jax.ShapeDtypeStruct((ELP,), jnp.float32),
    )(pred16)


def _bn_relu(y, g, b):
    mu = jnp.mean(y, axis=0, keepdims=True)
    var = jnp.mean((y - mu) ** 2, axis=0, keepdims=True)
    return jnp.maximum(g * (y - mu) * lax.rsqrt(var + 1e-5) + b, 0.0)


def _dot(a, b):
    return jnp.dot(a, b, preferred_element_type=jnp.float32)


def _stage_a_body(x_ref, w0t, b0r, g0r, be0r, b0t, h_ref, p_ref):
    h = _bn_relu(_dot(x_ref[...], w0t[...]) + b0r[...], g0r[...], be0r[...])
    h_ref[...] = h
    p_ref[...] = _dot(h, b0t[...])


def _layer_core(h, part_ref, dego_ref, at, skipt, bcv, gp, bp, wzt, bzr, wht, bhr):
    pa = part_ref[0, :N, :] + part_ref[1, :N, :]
    deg = dego_ref[0, :N, 0:1] + dego_ref[1, :N, 0:1]
    aggr = pa + deg * _dot(h, at) + _dot(h, skipt) + bcv
    p = _bn_relu(aggr, gp, bp)
    z = jax.nn.sigmoid(_dot(p, wzt) + bzr)
    ht = jnp.tanh(_dot(p, wht) + bhr)
    return (1.0 - z) * ht


def _stage_b_body(h_ref, part_ref, dego_ref, at, skipt, bcv, gp, bp,
                  wzt, bzr, wht, bhr, b1t, h1_ref, p1_ref):
    h1 = _layer_core(h_ref[...], part_ref, dego_ref, at[...], skipt[...],
                     bcv[...], gp[...], bp[...], wzt[...], bzr[...],
                     wht[...], bhr[...])
    h1_ref[...] = h1
    p1_ref[...] = _dot(h1, b1t[...])


def _stage_c_body(h_ref, part_ref, dego_ref, at, skipt, bcv, gp, bp,
                  wzt, bzr, wht, bhr, wdu, wdv, bdr, u_ref, v_ref):
    h2 = _layer_core(h_ref[...], part_ref, dego_ref, at[...], skipt[...],
                     bcv[...], gp[...], bp[...], wzt[...], bzr[...],
                     wht[...], bhr[...])
    u_ref[...] = _dot(h2, wdu[...]) + bdr[...]
    v_ref[...] = _dot(h2, wdv[...])


def _tc_call(body, out_shapes, *args):
    return pl.pallas_call(
        body,
        out_shape=[jax.ShapeDtypeStruct(s, jnp.float32) for s in out_shapes],
    )(*args)


def kernel(x, edge_index, edge_label_index, H_list, W0, b0, g0, be0, Wmsg,
           Wskip, bskip, bconv, gpl, bpl, Wz, bz, Wr, br, Wh, bh, Wdec, bdec):
    f32 = jnp.float32
    row = lambda v: v.reshape(1, HID)
    at0, at1 = Wmsg[0, :, :HID].T, Wmsg[1, :, :HID].T
    bt0, bt1 = Wmsg[0, :, HID:].T, Wmsg[1, :, HID:].T

    h0, p0 = _tc_call(_stage_a_body, [(N, HID), (N, HID)],
                      x, W0.T, row(b0), row(g0), row(be0), bt0)

    epad = EP - E
    iot = jax.lax.iota(jnp.int32, epad)
    src3 = jnp.concatenate(
        [edge_index[0].astype(jnp.int32), iot % N]
    ).reshape(NW, KCH, CHUNK)
    dst3 = jnp.concatenate(
        [edge_index[1].astype(jnp.int32), N + iot % (NACC - N)]
    ).reshape(NW, KCH, CHUNK)
    pad = jax.lax.iota(jnp.int32, ELP - EL) % N
    el0 = jnp.concatenate([edge_label_index[0].astype(jnp.int32), pad]).reshape(NW, KD, DCHUNK)
    el1 = jnp.concatenate([edge_label_index[1].astype(jnp.int32), pad]).reshape(NW, KD, DCHUNK)
    zrow = jnp.zeros((ROWS_T, HID), f32)
    zdeg = jnp.zeros((ROWS_T, DEGW), f32)
    ones = jnp.ones((CHUNK, DEGW), f32)

    dego = _degree(dst3, zdeg, ones)
    part0 = _spmm(p0, src3, dst3, zrow)

    h1, p1 = _tc_call(
        _stage_b_body, [(N, HID), (N, HID)],
        h0, part0, dego, at0, Wskip[0].T,
        row(bconv[0] + bskip[0]), row(gpl[0]), row(bpl[0]),
        Wz[0, :, :HID].T, row(bz[0]), Wh[0, :, :HID].T, row(bh[0]), bt1)

    part1 = _spmm(p1, src3, dst3, zrow)

    wdu = jnp.tile(Wdec[0, :HID].reshape(HID, 1), (1, DEGW))
    wdv = jnp.tile(Wdec[0, HID:].reshape(HID, 1), (1, DEGW))
    bdr = jnp.tile(bdec.reshape(1, 1), (1, DEGW))
    u16, v16 = _tc_call(
        _stage_c_body, [(N, DEGW), (N, DEGW)],
        h1, part1, dego, at1, Wskip[1].T,
        row(bconv[1] + bskip[1]), row(gpl[1]), row(bpl[1]),
        Wz[1, :, :HID].T, row(bz[1]), Wh[1, :, :HID].T, row(bh[1]),
        wdu, wdv, bdr)

    pred16 = _decode(u16, v16, el0, el1)
    pred = _compact(pred16.reshape(NW * OROWS, HID))
    return pred[:EL].reshape(EL, 1)


# speedup vs baseline: 1.0442x; 1.0442x over previous
import functools

import jax
import jax.numpy as jnp
from jax import lax
from jax.experimental import pallas as pl
from jax.experimental.pallas import tpu as pltpu
from jax.experimental.pallas import tpu_sc as plsc

N = 10000
HID = 128
E = 320000
EL = 100000
NW = 32
TILES = 16
CHUNK = 128
KCH = 80
EDGES_T = KCH * CHUNK
EP = NW * EDGES_T
NACC = 10240
ROWS_T = NACC // TILES
DEGW = 16
DCHUNK = 128
KD = 25
ELP_T = KD * DCHUNK
ELP = NW * ELP_T

_sc_mesh = plsc.VectorSubcoreMesh(core_axis_name="c", subcore_axis_name="s")


HALF = KCH // 2
PIPE = HALF // 2


@functools.partial(
    pl.kernel, mesh=_sc_mesh,
    out_type=jax.ShapeDtypeStruct((2, NACC, HID), jnp.float32),
    scratch_types=[
        pltpu.VMEM((HALF, CHUNK), jnp.int32),
        pltpu.VMEM((HALF, CHUNK), jnp.int32),
        pltpu.VMEM((CHUNK, HID), jnp.float32),
        pltpu.VMEM((CHUNK, HID), jnp.float32),
        pltpu.VMEM_SHARED((NACC, HID), jnp.float32),
        pltpu.SemaphoreType.DMA,
        pltpu.SemaphoreType.DMA,
    ],
)
def _spmm(p_h, src_h, dst_h, zrow_h, part_h, srcv, dstv, rows0, rows1,
          accsp, sem0, sem1):
    c = lax.axis_index("c")
    s = lax.axis_index("s")
    eb = c * TILES + s
    pltpu.sync_copy(zrow_h, accsp.at[pl.ds(s * ROWS_T, ROWS_T)])
    plsc.subcore_barrier()

    for half in (0, 1):
        pltpu.sync_copy(src_h.at[eb, pl.ds(half * HALF, HALF)], srcv)
        pltpu.sync_copy(dst_h.at[eb, pl.ds(half * HALF, HALF)], dstv)
        pltpu.async_copy(p_h.at[srcv.at[0]], rows0, sem0)

        def body(j2, carry):
            pltpu.async_copy(p_h.at[srcv.at[2 * j2 + 1]], rows1, sem1)
            pltpu.make_async_copy(p_h.at[srcv.at[0]], rows0, sem0).wait()
            pltpu.sync_copy(rows0, accsp.at[dstv.at[2 * j2]], add=True)

            @pl.when(j2 < PIPE - 1)
            def _prefetch():
                pltpu.async_copy(p_h.at[srcv.at[2 * j2 + 2]], rows0, sem0)

            pltpu.make_async_copy(p_h.at[srcv.at[0]], rows1, sem1).wait()
            pltpu.sync_copy(rows1, accsp.at[dstv.at[2 * j2 + 1]], add=True)
            return carry

        lax.fori_loop(0, PIPE, body, 0)

    plsc.subcore_barrier()
    pltpu.sync_copy(accsp.at[pl.ds(s * ROWS_T, ROWS_T)],
                    part_h.at[c, pl.ds(s * ROWS_T, ROWS_T)])


@functools.partial(
    pl.kernel, mesh=_sc_mesh,
    out_type=jax.ShapeDtypeStruct((2, NACC, DEGW), jnp.float32),
    compiler_params=pltpu.CompilerParams(use_tc_tiling_on_sc=False),
    scratch_types=[
        pltpu.VMEM((KCH, CHUNK), jnp.int32),
        pltpu.VMEM((CHUNK, DEGW), jnp.float32),
        pltpu.VMEM_SHARED((NACC, DEGW), jnp.float32),
    ],
)
def _degree(dst_h, zdeg_h, ones_h, dego_h, dstv, onesv, degsp):
    c = lax.axis_index("c")
    s = lax.axis_index("s")
    eb = c * TILES + s
    pltpu.sync_copy(dst_h.at[eb], dstv)
    pltpu.sync_copy(ones_h, onesv)
    pltpu.sync_copy(zdeg_h, degsp.at[pl.ds(s * ROWS_T, ROWS_T)])
    plsc.subcore_barrier()

    def body(j, carry):
        pltpu.sync_copy(onesv, degsp.at[dstv.at[j]], add=True)
        return carry

    lax.fori_loop(0, KCH, body, 0)
    plsc.subcore_barrier()
    pltpu.sync_copy(degsp.at[pl.ds(s * ROWS_T, ROWS_T)],
                    dego_h.at[c, pl.ds(s * ROWS_T, ROWS_T)])


KDP = KD // 2


OROWS = ELP_T * DEGW // HID


@functools.partial(
    pl.kernel, mesh=_sc_mesh,
    out_type=jax.ShapeDtypeStruct((NW, ELP_T, DEGW), jnp.float32),
    compiler_params=pltpu.CompilerParams(use_tc_tiling_on_sc=False),
    scratch_types=[
        pltpu.VMEM((KD, DCHUNK), jnp.int32),
        pltpu.VMEM((KD, DCHUNK), jnp.int32),
        pltpu.VMEM((ELP_T, DEGW), jnp.float32),
        pltpu.SemaphoreType.DMA,
        pltpu.SemaphoreType.DMA,
    ],
)
def _decode(u_h, v_h, el0_h, el1_h, out_h, i0v, i1v, outv, sem0, sem1):
    c = lax.axis_index("c")
    s = lax.axis_index("s")
    wid = c * TILES + s
    pltpu.sync_copy(el0_h.at[wid], i0v)
    pltpu.sync_copy(el1_h.at[wid], i1v)

    def oslice(j):
        return outv.at[pl.ds(j * DCHUNK, DCHUNK)]

    pltpu.async_copy(u_h.at[i0v.at[0]], oslice(0), sem0)

    def body(j2, carry):
        pltpu.async_copy(u_h.at[i0v.at[2 * j2 + 1]], oslice(2 * j2 + 1), sem1)
        pltpu.make_async_copy(u_h.at[i0v.at[0]], oslice(2 * j2), sem0).wait()
        pltpu.sync_copy(v_h.at[i1v.at[2 * j2]], oslice(2 * j2), add=True)
        pltpu.async_copy(u_h.at[i0v.at[2 * j2 + 2]], oslice(2 * j2 + 2), sem0)
        pltpu.make_async_copy(u_h.at[i0v.at[0]], oslice(2 * j2 + 1), sem1).wait()
        pltpu.sync_copy(v_h.at[i1v.at[2 * j2 + 1]], oslice(2 * j2 + 1), add=True)
        return carry

    lax.fori_loop(0, KDP, body, 0)
    pltpu.make_async_copy(u_h.at[i0v.at[0]], oslice(KD - 1), sem0).wait()
    pltpu.sync_copy(v_h.at[i1v.at[KD - 1]], oslice(KD - 1), add=True)
    pltpu.sync_copy(outv, out_h.at[wid])


CBLK = 10240


def _compact_body(pin_ref, out_ref):
    x = pin_ref[...]
    z = x.reshape(CBLK // 8, 8, DEGW)[:, :, 0]
    out_ref[...] = z.reshape(CBLK)


def _compact(pred16):
    return pl.pallas_call(
        _compact_body,
        grid=(ELP // CBLK,),
        in_specs=[pl.BlockSpec((CBLK // 8, HID), lambda t: (t, 0))],
        out_specs=pl.BlockSpec((CBLK,), lambda t: (t,)),
        out_shape=jax.ShapeDtypeStruct((ELP,), jnp.float32),
    )(pred16)


def _bn_relu(y, g, b):
    mu = jnp.mean(y, axis=0, keepdims=True)
    var = jnp.mean((y - mu) ** 2, axis=0, keepdims=True)
    return jnp.maximum(g * (y - mu) * lax.rsqrt(var + 1e-5) + b, 0.0)


def _dot(a, b):
    return jnp.dot(a, b, preferred_element_type=jnp.float32)


def _stage_a_body(x_ref, w0t, b0r, g0r, be0r, b0t, h_ref, p_ref):
    h = _bn_relu(_dot(x_ref[...], w0t[...]) + b0r[...], g0r[...], be0r[...])
    h_ref[...] = h
    p_ref[...] = _dot(h, b0t[...])


def _layer_core(h, part_ref, dego_ref, at, skipt, bcv, gp, bp, wzt, bzr, wht, bhr):
    pa = part_ref[0, :N, :] + part_ref[1, :N, :]
    deg = dego_ref[0, :N, 0:1] + dego_ref[1, :N, 0:1]
    aggr = pa + deg * _dot(h, at) + _dot(h, skipt) + bcv
    p = _bn_relu(aggr, gp, bp)
    z = jax.nn.sigmoid(_dot(p, wzt) + bzr)
    ht = jnp.tanh(_dot(p, wht) + bhr)
    return (1.0 - z) * ht


def _stage_b_body(h_ref, part_ref, dego_ref, at, skipt, bcv, gp, bp,
                  wzt, bzr, wht, bhr, b1t, h1_ref, p1_ref):
    h1 = _layer_core(h_ref[...], part_ref, dego_ref, at[...], skipt[...],
                     bcv[...], gp[...], bp[...], wzt[...], bzr[...],
                     wht[...], bhr[...])
    h1_ref[...] = h1
    p1_ref[...] = _dot(h1, b1t[...])


def _stage_c_body(h_ref, part_ref, dego_ref, at, skipt, bcv, gp, bp,
                  wzt, bzr, wht, bhr, wdu, wdv, bdr, u_ref, v_ref):
    h2 = _layer_core(h_ref[...], part_ref, dego_ref, at[...], skipt[...],
                     bcv[...], gp[...], bp[...], wzt[...], bzr[...],
                     wht[...], bhr[...])
    u_ref[...] = _dot(h2, wdu[...]) + bdr[...]
    v_ref[...] = _dot(h2, wdv[...])


def _tc_call(body, out_shapes, *args):
    return pl.pallas_call(
        body,
        out_shape=[jax.ShapeDtypeStruct(s, jnp.float32) for s in out_shapes],
    )(*args)


def kernel(x, edge_index, edge_label_index, H_list, W0, b0, g0, be0, Wmsg,
           Wskip, bskip, bconv, gpl, bpl, Wz, bz, Wr, br, Wh, bh, Wdec, bdec):
    f32 = jnp.float32
    row = lambda v: v.reshape(1, HID)
    at0, at1 = Wmsg[0, :, :HID].T, Wmsg[1, :, :HID].T
    bt0, bt1 = Wmsg[0, :, HID:].T, Wmsg[1, :, HID:].T

    h0, p0 = _tc_call(_stage_a_body, [(N, HID), (N, HID)],
                      x, W0.T, row(b0), row(g0), row(be0), bt0)

    epad = EP - E
    iot = jax.lax.iota(jnp.int32, epad)
    src3 = jnp.concatenate(
        [edge_index[0].astype(jnp.int32), iot % N]
    ).reshape(NW, KCH, CHUNK)
    dst3 = jnp.concatenate(
        [edge_index[1].astype(jnp.int32), N + iot % (NACC - N)]
    ).reshape(NW, KCH, CHUNK)
    pad = jax.lax.iota(jnp.int32, ELP - EL) % N
    el0 = jnp.concatenate([edge_label_index[0].astype(jnp.int32), pad]).reshape(NW, KD, DCHUNK)
    el1 = jnp.concatenate([edge_label_index[1].astype(jnp.int32), pad]).reshape(NW, KD, DCHUNK)
    zrow = jnp.zeros((ROWS_T, HID), f32)
    zdeg = jnp.zeros((ROWS_T, DEGW), f32)
    ones = jnp.ones((CHUNK, DEGW), f32)

    dego = _degree(dst3, zdeg, ones)
    part0 = _spmm(p0, src3, dst3, zrow)

    h1, p1 = _tc_call(
        _stage_b_body, [(N, HID), (N, HID)],
        h0, part0, dego, at0, Wskip[0].T,
        row(bconv[0] + bskip[0]), row(gpl[0]), row(bpl[0]),
        Wz[0, :, :HID].T, row(bz[0]), Wh[0, :, :HID].T, row(bh[0]), bt1)

    part1 = _spmm(p1, src3, dst3, zrow)

    wdu = jnp.tile(Wdec[0, :HID].reshape(HID, 1), (1, DEGW))
    wdv = jnp.tile(Wdec[0, HID:].reshape(HID, 1), (1, DEGW))
    bdr = jnp.tile(bdec.reshape(1, 1), (1, DEGW))
    u16, v16 = _tc_call(
        _stage_c_body, [(N, DEGW), (N, DEGW)],
        h1, part1, dego, at1, Wskip[1].T,
        row(bconv[1] + bskip[1]), row(gpl[1]), row(bpl[1]),
        Wz[1, :, :HID].T, row(bz[1]), Wh[1, :, :HID].T, row(bh[1]),
        wdu, wdv, bdr)

    pred16 = _decode(u16, v16, el0, el1)
    pred = _compact(pred16.reshape(NW * OROWS, HID))
    return pred[:EL].reshape(EL, 1)
